# Initial kernel scaffold; baseline (speedup 1.0000x reference)
import jax
import jax.numpy as jnp
from jax.experimental import pallas as pl


def kernel(input):
    raise NotImplementedError("write your pallas kernel here")


# speedup vs baseline: 5.7021x; 5.7021x over previous
import functools

import jax
import jax.numpy as jnp
from jax import lax
from jax.experimental import pallas as pl
from jax.experimental.pallas import tpu as pltpu
from jax.experimental.pallas import tpu_sc as plsc

L = 16
NITER = 30


def _sparsegen_rows(x_hbm, out_hbm, x_v, c_v, rows_per_w, n):
    nchunk = n // L
    info = plsc.get_sparse_core_info()
    nc = info.num_cores
    wid = lax.axis_index("s") * nc + lax.axis_index("c")
    base = wid * rows_per_w

    for r in range(rows_per_w):
        pltpu.sync_copy(x_hbm.at[base + r], x_v.at[pl.ds(r * n, n)])

    for r in range(rows_per_w):
        off = r * n

        def mx_body(i, mv):
            v = x_v[pl.ds(off + i * L, L)]
            return jnp.maximum(mv, v)

        mv = lax.fori_loop(1, nchunk, mx_body, x_v[pl.ds(off, L)])
        m = jnp.max(mv)
        lo0 = m - 1.0

        def cp_body(i, nk):
            v = x_v[pl.ds(off + i * L, L)]
            c_v[pl.ds(nk * L, L)] = v
            keep = jnp.any(v > lo0)
            return nk + keep.astype(jnp.int32)

        nk = lax.fori_loop(0, nchunk, cp_body, jnp.int32(0))

        def bi_body(_, carry):
            lo, hi = carry
            mid = 0.5 * (lo + hi)

            def s_body(i, acc):
                v = c_v[pl.ds(i * L, L)]
                return acc + jnp.maximum(v - mid, 0.0)

            acc = lax.fori_loop(0, nk, s_body, jnp.zeros((L,), jnp.float32))
            s = jnp.sum(acc)
            gt = s > 1.0
            return jnp.where(gt, mid, lo), jnp.where(gt, hi, mid)

        lo, hi = lax.fori_loop(0, NITER, bi_body, (lo0, m))
        tau = 0.5 * (lo + hi)

        def o_body(i, carry):
            v = x_v[pl.ds(off + i * L, L)]
            x_v[pl.ds(off + i * L, L)] = jnp.maximum(v - tau, 0.0)
            return carry

        lax.fori_loop(0, nchunk, o_body, jnp.int32(0))

    for r in range(rows_per_w):
        pltpu.sync_copy(x_v.at[pl.ds(r * n, n)], out_hbm.at[base + r])


def kernel(input):
    orig_shape = input.shape
    x = input.reshape(-1, input.shape[-1])
    rows, n = x.shape

    info = plsc.get_sparse_core_info()
    nw = info.num_cores * info.num_subcores
    rows_per_w = rows // nw
    assert rows_per_w * nw == rows and n % L == 0

    mesh = plsc.VectorSubcoreMesh(core_axis_name="c", subcore_axis_name="s")
    body = functools.partial(_sparsegen_rows, rows_per_w=rows_per_w, n=n)
    f = functools.partial(
        pl.kernel,
        mesh=mesh,
        out_type=jax.ShapeDtypeStruct((rows, n), jnp.float32),
        scratch_types=[
            pltpu.VMEM((rows_per_w * n,), jnp.float32),
            pltpu.VMEM((n,), jnp.float32),
        ],
        compiler_params=pltpu.CompilerParams(needs_layout_passes=False),
    )(body)
    out = f(x)
    return out.reshape(orig_shape)
